# baseline (speedup 1.0000x reference)
import jax
import jax.numpy as jnp
from jax import lax
from jax.experimental import pallas as pl
from jax.experimental.pallas import tpu as pltpu

D_MODEL = 1024
NUM_EXPERTS = 8
SEQ = 2048
NBUF = 3


def _moe_body(x_ref, shared_W_ref, shared_b_ref, router_W_ref,
              router_b_ref, ew_hbm, eb_ref, out_ref,
              coeff_ref, xbf_ref, wbuf_ref, sems):
    u = pl.program_id(0)

    @pl.when(u == 0)
    def _():
        for k in range(NBUF):
            pltpu.make_async_copy(ew_hbm.at[k], wbuf_ref.at[k],
                                  sems.at[k]).start()
        xb = x_ref[...]
        xb16 = xb.astype(jnp.bfloat16)
        xbf_ref[...] = xb16
        scores = lax.dot_general(xb, router_W_ref[...],
                                 (((1,), (1,)), ((), ())),
                                 preferred_element_type=jnp.float32)
        scores = scores + router_b_ref[...]
        eidx = lax.broadcasted_iota(jnp.int32, scores.shape, 1)
        m0 = jnp.max(scores, axis=-1, keepdims=True)
        a0 = jnp.min(jnp.where(scores == m0, eidx, NUM_EXPERTS), axis=-1,
                     keepdims=True)
        masked = jnp.where(eidx == a0, -jnp.inf, scores)
        m1 = jnp.max(masked, axis=-1, keepdims=True)
        a1 = jnp.min(jnp.where(masked == m1, eidx, NUM_EXPERTS), axis=-1,
                     keepdims=True)
        z = jnp.exp(m1 - m0)
        w0 = 1.0 / (1.0 + z)
        w1 = z * w0
        coeff = (jnp.where(eidx == a0, w0, 0.0)
                 + jnp.where(eidx == a1, w1, 0.0))
        coeff_ref[...] = coeff
        so = lax.dot_general(xb16, shared_W_ref[...].astype(jnp.bfloat16),
                             (((1,), (1,)), ((), ())),
                             preferred_element_type=jnp.float32)
        bias_mix = lax.dot_general(coeff, eb_ref[...],
                                   (((1,), (0,)), ((), ())),
                                   preferred_element_type=jnp.float32)
        out_ref[...] = so + shared_b_ref[...] + bias_mix

    @pl.when(u > 0)
    def _():
        e = u - 1
        slot = lax.rem(e, NBUF)
        pltpu.make_async_copy(ew_hbm.at[e], wbuf_ref.at[slot],
                              sems.at[slot]).wait()
        call = coeff_ref[...]
        lane = lax.broadcasted_iota(jnp.int32, call.shape, 1)
        coeff = jnp.sum(jnp.where(lane == e, call, 0.0), axis=1,
                        keepdims=True)
        eo = lax.dot_general(xbf_ref[...],
                             wbuf_ref[slot].astype(jnp.bfloat16),
                             (((1,), (1,)), ((), ())),
                             preferred_element_type=jnp.float32)
        out_ref[...] += coeff * eo
        nxt = e + NBUF

        @pl.when(nxt < NUM_EXPERTS)
        def _():
            nslot = lax.rem(nxt, NBUF)
            pltpu.make_async_copy(ew_hbm.at[nxt], wbuf_ref.at[nslot],
                                  sems.at[nslot]).start()


@jax.jit
def kernel(x, shared_W, shared_b, router_W, router_b, expert_W, expert_b):
    B, S, D = x.shape
    x2 = x.reshape(S, D)

    out = pl.pallas_call(
        _moe_body,
        grid=(NUM_EXPERTS + 1,),
        in_specs=[
            pl.BlockSpec((S, D), lambda u: (0, 0)),
            pl.BlockSpec((D, D), lambda u: (0, 0)),
            pl.BlockSpec((1, D), lambda u: (0, 0)),
            pl.BlockSpec((NUM_EXPERTS, D), lambda u: (0, 0)),
            pl.BlockSpec((1, NUM_EXPERTS), lambda u: (0, 0)),
            pl.BlockSpec(memory_space=pl.ANY),
            pl.BlockSpec((NUM_EXPERTS, D), lambda u: (0, 0)),
        ],
        out_specs=pl.BlockSpec((S, D), lambda u: (0, 0)),
        out_shape=jax.ShapeDtypeStruct((S, D), jnp.float32),
        scratch_shapes=[pltpu.VMEM((S, NUM_EXPERTS), jnp.float32),
                        pltpu.VMEM((S, D), jnp.bfloat16),
                        pltpu.VMEM((NBUF, D, D), jnp.float32),
                        pltpu.SemaphoreType.DMA((NBUF,))],
    )(x2, shared_W, shared_b.reshape(1, D),
      router_W, router_b.reshape(1, NUM_EXPERTS),
      expert_W, expert_b)
    return out.reshape(B, S, D)
